# Initial kernel scaffold; baseline (speedup 1.0000x reference)
import jax
import jax.numpy as jnp
from jax.experimental import pallas as pl


def kernel(control_pts, weights, u_spline_space, v_spline_space):
    raise NotImplementedError("write your pallas kernel here")


# speedup vs baseline: 34.4684x; 34.4684x over previous
import jax
import jax.numpy as jnp
from jax.experimental import pallas as pl
from jax.experimental.pallas import tpu as pltpu

_DEG = 3
_NC = 128
_NSEG = _NC - _DEG
_N = 1024
_TILE = 128
_PREC = jax.lax.Precision.HIGHEST


def _span_basis(t2):
    kj = jax.lax.broadcasted_iota(jnp.int32, (_NC, _N), 0).astype(jnp.float32)
    pred = (t2 - kj / float(_NSEG)) > 1e-8
    cnt = jnp.sum(pred.astype(jnp.int32), axis=0, keepdims=True)
    span = jnp.maximum(cnt - 1, 0) + _DEG
    span_f = span.astype(jnp.float32)

    N = [jnp.ones_like(t2), None, None, None]
    for k in range(1, _DEG + 1):
        saved = jnp.zeros_like(t2)
        for r in range(k):
            V1 = jnp.clip((span_f + float(r - 2)) / float(_NSEG), 0.0, 1.0)
            V2 = jnp.clip((span_f + float(r - k - 2)) / float(_NSEG), 0.0, 1.0)
            denom = (V1 - t2) + (t2 - V2)
            temp = jnp.where(denom == 0.0, jnp.full_like(t2, 0.0001),
                             N[r] / denom)
            N[r] = saved + (V1 - t2) * temp
            saved = (t2 - V2) * temp
        N[k] = saved
    return span, N


def _basis_mat_t(span, N):
    m = jax.lax.broadcasted_iota(jnp.int32, (_NC, _N), 0)
    base = span - _DEG
    acc = jnp.zeros((_NC, _N), jnp.float32)
    for l in range(_DEG + 1):
        acc = acc + jnp.where(m == base + l, N[l], 0.0)
    return acc


def _body(cp_ref, w_ref, u_ref, v_ref, out_ref, but_ref, t_ref):
    i = pl.program_id(0)

    @pl.when(i == 0)
    def _setup():
        sv, Nv = _span_basis(v_ref[...])
        bvt = _basis_mat_t(sv, Nv)
        wmat = w_ref[...]
        for c in range(3):
            t_ref[c] = jnp.dot(cp_ref[c] * wmat, bvt,
                               preferred_element_type=jnp.float32,
                               precision=_PREC)
        t_ref[3] = jnp.dot(wmat, bvt,
                           preferred_element_type=jnp.float32, precision=_PREC)
        su, Nu = _span_basis(u_ref[...])
        but_ref[...] = _basis_mat_t(su, Nu)

    but_t = but_ref[:, pl.ds(i * _TILE, _TILE)]
    r = [jax.lax.dot_general(but_t, t_ref[c], (((0,), (0,)), ((), ())),
                             preferred_element_type=jnp.float32,
                             precision=_PREC)
         for c in range(4)]
    winv = 1.0 / r[3]
    for c in range(3):
        out_ref[c] = r[c] * winv


def kernel(control_pts, weights, u_spline_space, v_spline_space):
    cp = jnp.transpose(control_pts[0], (2, 0, 1))
    w = weights[0, :, :, 0]
    u2 = jnp.sort(u_spline_space)[None, :]
    v2 = v_spline_space[None, :]

    out2d = pl.pallas_call(
        _body,
        grid=(_N // _TILE,),
        in_specs=[
            pl.BlockSpec((3, _NC, _NC), lambda i: (0, 0, 0)),
            pl.BlockSpec((_NC, _NC), lambda i: (0, 0)),
            pl.BlockSpec((1, _N), lambda i: (0, 0)),
            pl.BlockSpec((1, _N), lambda i: (0, 0)),
        ],
        out_specs=pl.BlockSpec((3, _TILE, _N), lambda i: (0, i, 0)),
        out_shape=jax.ShapeDtypeStruct((3, _N, _N), jnp.float32),
        scratch_shapes=[
            pltpu.VMEM((_NC, _N), jnp.float32),
            pltpu.VMEM((4, _NC, _N), jnp.float32),
        ],
    )(cp, w, u2, v2)
    return jnp.transpose(out2d, (1, 2, 0))[None]


# speedup vs baseline: 46.1668x; 1.3394x over previous
import jax
import jax.numpy as jnp
from jax.experimental import pallas as pl
from jax.experimental.pallas import tpu as pltpu

_DEG = 3
_NC = 128
_NSEG = _NC - _DEG
_N = 1024
_TILE = 128
_PREC = jax.lax.Precision.HIGHEST


def _span_basis(t2):
    kj = jax.lax.broadcasted_iota(jnp.int32, (_NC, _N), 0).astype(jnp.float32)
    pred = (t2 - kj / float(_NSEG)) > 1e-8
    cnt = jnp.sum(pred.astype(jnp.int32), axis=0, keepdims=True)
    span = jnp.maximum(cnt - 1, 0) + _DEG
    span_f = span.astype(jnp.float32)

    N = [jnp.ones_like(t2), None, None, None]
    for k in range(1, _DEG + 1):
        saved = jnp.zeros_like(t2)
        for r in range(k):
            V1 = jnp.clip((span_f + float(r - 2)) / float(_NSEG), 0.0, 1.0)
            V2 = jnp.clip((span_f + float(r - k - 2)) / float(_NSEG), 0.0, 1.0)
            denom = (V1 - t2) + (t2 - V2)
            temp = jnp.where(denom == 0.0, jnp.full_like(t2, 0.0001),
                             N[r] / denom)
            N[r] = saved + (V1 - t2) * temp
            saved = (t2 - V2) * temp
        N[k] = saved
    return span, N


def _basis_mat_t(span, N):
    m = jax.lax.broadcasted_iota(jnp.int32, (_NC, _N), 0)
    base = span - _DEG
    acc = jnp.zeros((_NC, _N), jnp.float32)
    for l in range(_DEG + 1):
        acc = acc + jnp.where(m == base + l, N[l], 0.0)
    return acc


def _split(x):
    hi = x.astype(jnp.bfloat16)
    lo = (x - hi.astype(jnp.float32)).astype(jnp.bfloat16)
    return hi, lo


def _body(cp_ref, w_ref, u_ref, v_ref, out_ref,
          buthi_ref, butlo_ref, thi_ref, tlo_ref):
    i = pl.program_id(0)

    @pl.when(i == 0)
    def _setup():
        sv, Nv = _span_basis(v_ref[...])
        bvt = _basis_mat_t(sv, Nv)
        wmat = w_ref[...]
        for c in range(4):
            ctrl_c = wmat if c == 3 else cp_ref[c] * wmat
            tc = jnp.dot(ctrl_c, bvt, preferred_element_type=jnp.float32,
                         precision=_PREC)
            thi_ref[c], tlo_ref[c] = _split(tc)
        su, Nu = _span_basis(u_ref[...])
        but = _basis_mat_t(su, Nu)
        buthi_ref[...], butlo_ref[...] = _split(but)

    sl = pl.ds(i * _TILE, _TILE)
    bhi, blo = buthi_ref[:, sl], butlo_ref[:, sl]
    dims = (((0,), (0,)), ((), ()))

    def dot3(c):
        acc = jax.lax.dot_general(bhi, thi_ref[c], dims,
                                  preferred_element_type=jnp.float32)
        acc += jax.lax.dot_general(bhi, tlo_ref[c], dims,
                                   preferred_element_type=jnp.float32)
        acc += jax.lax.dot_general(blo, thi_ref[c], dims,
                                   preferred_element_type=jnp.float32)
        return acc

    r = [dot3(c) for c in range(4)]
    winv = 1.0 / r[3]
    for c in range(3):
        out_ref[c] = r[c] * winv


def kernel(control_pts, weights, u_spline_space, v_spline_space):
    cp = jnp.transpose(control_pts[0], (2, 0, 1))
    w = weights[0, :, :, 0]
    u2 = jnp.sort(u_spline_space)[None, :]
    v2 = v_spline_space[None, :]

    out2d = pl.pallas_call(
        _body,
        grid=(_N // _TILE,),
        in_specs=[
            pl.BlockSpec((3, _NC, _NC), lambda i: (0, 0, 0)),
            pl.BlockSpec((_NC, _NC), lambda i: (0, 0)),
            pl.BlockSpec((1, _N), lambda i: (0, 0)),
            pl.BlockSpec((1, _N), lambda i: (0, 0)),
        ],
        out_specs=pl.BlockSpec((3, _TILE, _N), lambda i: (0, i, 0)),
        out_shape=jax.ShapeDtypeStruct((3, _N, _N), jnp.float32),
        scratch_shapes=[
            pltpu.VMEM((_NC, _N), jnp.bfloat16),
            pltpu.VMEM((_NC, _N), jnp.bfloat16),
            pltpu.VMEM((4, _NC, _N), jnp.bfloat16),
            pltpu.VMEM((4, _NC, _N), jnp.bfloat16),
        ],
    )(cp, w, u2, v2)
    return jnp.transpose(out2d, (1, 2, 0))[None]
